# baseline (speedup 1.0000x reference)
import functools

import jax
import jax.numpy as jnp
from jax import lax
from jax.experimental import pallas as pl
from jax.experimental.pallas import tpu as pltpu
from jax.experimental.pallas import tpu_sc as plsc

K = 8192
DIM = 256
MEM_DIM = 1024
Q_DIM = 1024
B = 32
S = 2048

NB = 16
KB = K // NB
CHUNK = 8
MARGIN = 0.05
BIG = 3.4e38

_BF = jnp.bfloat16
_F32 = jnp.float32


def _bdot(a, b, dims):
    return lax.dot_general(a.astype(_BF), b.astype(_BF), (dims, ((), ())),
                           preferred_element_type=_F32)


def _attn_vq_body(m0_ref, m1_ref, m2_ref, m3_ref,
                  lens_ref, q_ref, ws_ref, wo_ref, emb_ref,
                  d_ref, v_ref, qp_ref, e2_ref, bm_ref):
    b = pl.program_id(0)

    @pl.when(b == 0)
    def _():
        e = emb_ref[...]
        e2_ref[...] = jnp.sum(e * e, axis=1).reshape(1, K)
        qp_ref[...] = _bdot(q_ref[...], ws_ref[...], (((1,), (1,))))
        d_ref[...] = jnp.full((B, K), BIG, _F32)

    mem_bf = jnp.concatenate(
        [m0_ref[0].astype(_BF), m1_ref[0].astype(_BF),
         m2_ref[0].astype(_BF), m3_ref[0].astype(_BF)], axis=0)
    qp = qp_ref[pl.ds(b, 1), :]
    scores = lax.dot_general(qp.astype(_BF), mem_bf, (((1,), (1,)), ((), ())),
                             preferred_element_type=_F32)
    pos = lax.broadcasted_iota(jnp.int32, (1, S), 1)
    scores = jnp.where(pos < lens_ref[b], scores, -1e9)
    m = jnp.max(scores)
    ex = jnp.exp(scores - m)
    alpha = ex / jnp.sum(ex)
    context = lax.dot_general(alpha.astype(_BF), mem_bf,
                              (((1,), (0,)), ((), ())),
                              preferred_element_type=_F32)
    qrow = q_ref[pl.ds(b, 1), :]
    cat = jnp.concatenate([context, qrow], axis=1)
    V = jnp.tanh(_bdot(cat, wo_ref[...], (((1,), (0,)))))
    v_ref[pl.ds(b, 1), :] = V

    def process_chunk(c):
        ipc = lax.dot_general(v_ref[pl.ds(c * CHUNK, CHUNK), :], emb_ref[...],
                              (((1,), (1,)), ((), ())),
                              precision=lax.Precision.HIGHEST,
                              preferred_element_type=_F32)
        dfast = e2_ref[...] - 2.0 * ipc
        bm_ref[pl.ds(c * CHUNK, CHUNK), :] = jnp.min(
            dfast.reshape(CHUNK, NB, KB), axis=2)

        def refine(b2, carry):
            bmr = bm_ref[pl.ds(b2, 1), :]
            mn = jnp.min(bmr)
            jb = jnp.argmin(bmr).astype(jnp.int32)
            cnt = jnp.sum((bmr < mn + MARGIN).astype(jnp.int32))
            Vb = v_ref[pl.ds(b2, 1), :]

            @pl.when(cnt == 1)
            def _():
                rows = emb_ref[pl.ds(jb * KB, KB), :]
                diffb = Vb - rows
                db = jnp.sum(diffb * diffb, axis=1)
                d_ref[pl.ds(b2, 1), pl.ds(jb * KB, KB)] = db.reshape(1, KB)

            @pl.when(cnt > 1)
            def _():
                diffa = Vb - emb_ref[...]
                da = jnp.sum(diffa * diffa, axis=1)
                d_ref[pl.ds(b2, 1), :] = da.reshape(1, K)

            return carry

        lax.fori_loop(c * CHUNK, (c + 1) * CHUNK, refine, 0)

    for c in range(B // CHUNK):
        trigger = (c + 1) * CHUNK if (c + 1) * CHUNK < B else B - 1
        pl.when(b == trigger)(functools.partial(process_chunk, c))


def _attn_vq(input_memory, input_lens, init_query, W_score, W_out, emb):
    return pl.pallas_call(
        _attn_vq_body,
        grid=(B,),
        in_specs=[
            pl.BlockSpec((1, S // 4, MEM_DIM), lambda b: (b, 0, 0)),
            pl.BlockSpec((1, S // 4, MEM_DIM), lambda b: (b, 1, 0)),
            pl.BlockSpec((1, S // 4, MEM_DIM), lambda b: (b, 2, 0)),
            pl.BlockSpec((1, S // 4, MEM_DIM), lambda b: (b, 3, 0)),
            pl.BlockSpec(memory_space=pltpu.SMEM),
            pl.BlockSpec((B, Q_DIM), lambda b: (0, 0)),
            pl.BlockSpec((MEM_DIM, Q_DIM), lambda b: (0, 0)),
            pl.BlockSpec((MEM_DIM + Q_DIM, DIM), lambda b: (0, 0)),
            pl.BlockSpec((K, DIM), lambda b: (0, 0)),
        ],
        out_specs=[
            pl.BlockSpec((B, K), lambda b: (0, 0)),
            pl.BlockSpec((B, DIM), lambda b: (0, 0)),
        ],
        out_shape=[
            jax.ShapeDtypeStruct((B, K), _F32),
            jax.ShapeDtypeStruct((B, DIM), _F32),
        ],
        scratch_shapes=[
            pltpu.VMEM((B, MEM_DIM), _F32),
            pltpu.VMEM((1, K), _F32),
            pltpu.VMEM((B, NB), _F32),
        ],
    )(input_memory, input_memory, input_memory, input_memory,
      input_lens, init_query, W_score, W_out, emb)


def _lane_perm(x, perm):
    return lax.gather(
        x, perm.reshape(16, 1),
        lax.GatherDimensionNumbers(offset_dims=(), collapsed_slice_dims=(0,),
                                   start_index_map=(0,)),
        (1,), mode=lax.GatherScatterMode.PROMISE_IN_BOUNDS)


def _lane_reduce(x, op):
    lane = lax.iota(jnp.int32, 16)
    for k in (1, 2, 4, 8):
        x = op(x, _lane_perm(x, lane ^ k))
    return x


def _sc_body(d_hbm, emb_hbm, v_hbm, coll_hbm, diff_hbm,
             d_v, wq_v, v_v, out_v, sem):
    info = plsc.get_sparse_core_info()
    nc = info.num_cores
    wid = lax.axis_index("s") * nc + lax.axis_index("c")

    pltpu.sync_copy(d_hbm.at[wid], d_v)
    pltpu.sync_copy(v_hbm.at[wid], v_v)

    lane = lax.iota(jnp.int32, 16)
    big = jnp.float32(3.4e38)

    def amin_step(j, carry):
        vmin, imin = carry
        chunk = d_v[pl.ds(j * 16, 16)]
        upd = chunk < vmin
        return (jnp.where(upd, chunk, vmin),
                jnp.where(upd, lane + j * 16, imin))

    vmin0 = jnp.full((16,), big, _F32)
    imin0 = jnp.full((16,), 2**30, jnp.int32)
    vmin, imin = lax.fori_loop(0, K // 16, amin_step, (vmin0, imin0))
    gmin = _lane_reduce(vmin, jnp.minimum)
    gidx = _lane_reduce(jnp.where(vmin == gmin, imin, 2**30), jnp.minimum)

    pltpu.async_copy(emb_hbm.at[gidx], wq_v, sem).wait()

    acc = jnp.zeros((16,), _F32)
    for c in range(DIM // 16):
        t = wq_v[0, pl.ds(c * 16, 16)] - v_v[pl.ds(c * 16, 16)]
        acc = acc + t * t
    diff = _lane_reduce(acc, jnp.add)

    pltpu.sync_copy(wq_v.at[0], coll_hbm.at[wid, 0])
    out_v[...] = diff
    pltpu.sync_copy(out_v, diff_hbm.at[wid])


def _sc_select(d, emb, V):
    mesh = plsc.VectorSubcoreMesh(core_axis_name="c", subcore_axis_name="s")
    k = pl.kernel(
        _sc_body,
        out_type=(jax.ShapeDtypeStruct((B, 1, DIM), _F32),
                  jax.ShapeDtypeStruct((B, 16), _F32)),
        mesh=mesh,
        scratch_types=[
            pltpu.VMEM((K,), _F32),
            pltpu.VMEM((16, DIM), _F32),
            pltpu.VMEM((DIM,), _F32),
            pltpu.VMEM((16,), _F32),
            pltpu.SemaphoreType.DMA,
        ],
    )
    return k(d, emb, V)


def kernel(input_memory, input_lens, init_query, W_score, W_out, emb):
    lens32 = input_lens.astype(jnp.int32)
    d, V = _attn_vq(input_memory, lens32, init_query, W_score, W_out, emb)
    collected, diffpad = _sc_select(d, emb, V)
    diff = diffpad[:, 0]
    return (collected, diff, diff)


# speedup vs baseline: 5.4857x; 5.4857x over previous
import functools

import jax
import jax.numpy as jnp
from jax import lax
from jax.experimental import pallas as pl
from jax.experimental.pallas import tpu as pltpu
from jax.experimental.pallas import tpu_sc as plsc

K = 8192
DIM = 256
MEM_DIM = 1024
Q_DIM = 1024
B = 32
S = 2048

NB = 16
KB = K // NB
MARGIN = 0.05
BIG = 3.4e38

_BF = jnp.bfloat16
_F32 = jnp.float32


def _bdot(a, b, dims):
    return lax.dot_general(a.astype(_BF), b.astype(_BF), (dims, ((), ())),
                           preferred_element_type=_F32)


def _attn_vq_body(m0_ref, m1_ref, m2_ref, m3_ref,
                  lens_ref, q_ref, ws_ref, wo_ref, emb_ref,
                  d_ref, v_ref, qp_ref, e2_ref, bm_ref):
    b = pl.program_id(0)

    @pl.when(b == 0)
    def _():
        e = emb_ref[...]
        e2_ref[...] = jnp.sum(e * e, axis=1).reshape(1, K)
        qp_ref[...] = _bdot(q_ref[...], ws_ref[...], (((1,), (1,))))
        d_ref[...] = jnp.full((B, K), BIG, _F32)

    mem_bf = jnp.concatenate(
        [m0_ref[0].astype(_BF), m1_ref[0].astype(_BF),
         m2_ref[0].astype(_BF), m3_ref[0].astype(_BF)], axis=0)
    qp = qp_ref[pl.ds(b, 1), :]
    scores = lax.dot_general(qp.astype(_BF), mem_bf, (((1,), (1,)), ((), ())),
                             preferred_element_type=_F32)
    pos = lax.broadcasted_iota(jnp.int32, (1, S), 1)
    scores = jnp.where(pos < lens_ref[b], scores, -1e9)
    m = jnp.max(scores)
    ex = jnp.exp(scores - m)
    alpha = ex / jnp.sum(ex)
    context = lax.dot_general(alpha.astype(_BF), mem_bf,
                              (((1,), (0,)), ((), ())),
                              preferred_element_type=_F32)
    qrow = q_ref[pl.ds(b, 1), :]
    cat = jnp.concatenate([context, qrow], axis=1)
    V = jnp.tanh(_bdot(cat, wo_ref[...], (((1,), (0,)))))
    v_ref[pl.ds(b, 1), :] = V

    @pl.when(b == B - 1)
    def _():
        Vall = v_ref[...]
        ip = lax.dot_general(Vall, emb_ref[...], (((1,), (1,)), ((), ())),
                             precision=lax.Precision.HIGHEST,
                             preferred_element_type=_F32)
        dfast = e2_ref[...] - 2.0 * ip
        bm_ref[...] = jnp.min(dfast.reshape(B, NB, KB), axis=2)

        def refine(b2, carry):
            bmr = bm_ref[pl.ds(b2, 1), :]
            mn = jnp.min(bmr)
            jb = jnp.argmin(bmr).astype(jnp.int32)
            cnt = jnp.sum((bmr < mn + MARGIN).astype(jnp.int32))
            Vb = v_ref[pl.ds(b2, 1), :]

            @pl.when(cnt == 1)
            def _():
                rows = emb_ref[pl.ds(jb * KB, KB), :]
                diffb = Vb - rows
                db = jnp.sum(diffb * diffb, axis=1)
                d_ref[pl.ds(b2, 1), pl.ds(jb * KB, KB)] = db.reshape(1, KB)

            @pl.when(cnt > 1)
            def _():
                diffa = Vb - emb_ref[...]
                da = jnp.sum(diffa * diffa, axis=1)
                d_ref[pl.ds(b2, 1), :] = da.reshape(1, K)

            return carry

        lax.fori_loop(0, B, refine, 0)


def _attn_vq(input_memory, input_lens, init_query, W_score, W_out, emb):
    return pl.pallas_call(
        _attn_vq_body,
        grid=(B,),
        in_specs=[
            pl.BlockSpec((1, S // 4, MEM_DIM), lambda b: (b, 0, 0)),
            pl.BlockSpec((1, S // 4, MEM_DIM), lambda b: (b, 1, 0)),
            pl.BlockSpec((1, S // 4, MEM_DIM), lambda b: (b, 2, 0)),
            pl.BlockSpec((1, S // 4, MEM_DIM), lambda b: (b, 3, 0)),
            pl.BlockSpec(memory_space=pltpu.SMEM),
            pl.BlockSpec((B, Q_DIM), lambda b: (0, 0)),
            pl.BlockSpec((MEM_DIM, Q_DIM), lambda b: (0, 0)),
            pl.BlockSpec((MEM_DIM + Q_DIM, DIM), lambda b: (0, 0)),
            pl.BlockSpec((K, DIM), lambda b: (0, 0)),
        ],
        out_specs=[
            pl.BlockSpec((B, K), lambda b: (0, 0)),
            pl.BlockSpec((B, DIM), lambda b: (0, 0)),
        ],
        out_shape=[
            jax.ShapeDtypeStruct((B, K), _F32),
            jax.ShapeDtypeStruct((B, DIM), _F32),
        ],
        scratch_shapes=[
            pltpu.VMEM((B, MEM_DIM), _F32),
            pltpu.VMEM((1, K), _F32),
            pltpu.VMEM((B, NB), _F32),
        ],
    )(input_memory, input_memory, input_memory, input_memory,
      input_lens, init_query, W_score, W_out, emb)


def _lane_perm(x, perm):
    return lax.gather(
        x, perm.reshape(16, 1),
        lax.GatherDimensionNumbers(offset_dims=(), collapsed_slice_dims=(0,),
                                   start_index_map=(0,)),
        (1,), mode=lax.GatherScatterMode.PROMISE_IN_BOUNDS)


def _lane_reduce(x, op):
    lane = lax.iota(jnp.int32, 16)
    for k in (1, 2, 4, 8):
        x = op(x, _lane_perm(x, lane ^ k))
    return x


def _sc_body(d_hbm, emb_hbm, v_hbm, coll_hbm, diff_hbm,
             d_v, wq_v, v_v, out_v, sem):
    info = plsc.get_sparse_core_info()
    nc = info.num_cores
    wid = lax.axis_index("s") * nc + lax.axis_index("c")

    pltpu.sync_copy(d_hbm.at[wid], d_v)
    pltpu.sync_copy(v_hbm.at[wid], v_v)

    lane = lax.iota(jnp.int32, 16)
    big = jnp.float32(3.4e38)

    def amin_step(j, carry):
        vmin, imin = carry
        chunk = d_v[pl.ds(j * 16, 16)]
        upd = chunk < vmin
        return (jnp.where(upd, chunk, vmin),
                jnp.where(upd, lane + j * 16, imin))

    vmin0 = jnp.full((16,), big, _F32)
    imin0 = jnp.full((16,), 2**30, jnp.int32)
    vmin, imin = lax.fori_loop(0, K // 16, amin_step, (vmin0, imin0))
    gmin = _lane_reduce(vmin, jnp.minimum)
    gidx = _lane_reduce(jnp.where(vmin == gmin, imin, 2**30), jnp.minimum)

    pltpu.async_copy(emb_hbm.at[gidx], wq_v, sem).wait()

    acc = jnp.zeros((16,), _F32)
    for c in range(DIM // 16):
        t = wq_v[0, pl.ds(c * 16, 16)] - v_v[pl.ds(c * 16, 16)]
        acc = acc + t * t
    diff = _lane_reduce(acc, jnp.add)

    pltpu.sync_copy(wq_v.at[0], coll_hbm.at[wid, 0])
    out_v[...] = diff
    pltpu.sync_copy(out_v, diff_hbm.at[wid])


def _sc_select(d, emb, V):
    mesh = plsc.VectorSubcoreMesh(core_axis_name="c", subcore_axis_name="s")
    k = pl.kernel(
        _sc_body,
        out_type=(jax.ShapeDtypeStruct((B, 1, DIM), _F32),
                  jax.ShapeDtypeStruct((B, 16), _F32)),
        mesh=mesh,
        scratch_types=[
            pltpu.VMEM((K,), _F32),
            pltpu.VMEM((16, DIM), _F32),
            pltpu.VMEM((DIM,), _F32),
            pltpu.VMEM((16,), _F32),
            pltpu.SemaphoreType.DMA,
        ],
    )
    return k(d, emb, V)


def kernel(input_memory, input_lens, init_query, W_score, W_out, emb):
    lens32 = input_lens.astype(jnp.int32)
    d, V = _attn_vq(input_memory, lens32, init_query, W_score, W_out, emb)
    collected, diffpad = _sc_select(d, emb, V)
    diff = diffpad[:, 0]
    return (collected, diff, diff)
